# Initial kernel scaffold; baseline (speedup 1.0000x reference)
import jax
import jax.numpy as jnp
from jax.experimental import pallas as pl


def kernel(x, token_table, pos_table):
    raise NotImplementedError("write your pallas kernel here")


# speedup vs baseline: 3.9584x; 3.9584x over previous
import functools

import jax
import jax.numpy as jnp
from jax import lax
from jax.experimental import pallas as pl
from jax.experimental.pallas import tpu as pltpu
from jax.experimental.pallas import tpu_sc as plsc

NC = 2
NS = 16
LANES = 16


def _make_sc_kernel(B, S, V, D):
    n_tiles = NC * NS
    seqs_per_tile = B // n_tiles
    mesh = plsc.VectorSubcoreMesh(core_axis_name="c", subcore_axis_name="s")

    @functools.partial(
        pl.kernel,
        out_type=jax.ShapeDtypeStruct((B * S, D), jnp.float32),
        mesh=mesh,
        scratch_types=[
            pltpu.VMEM((S,), jnp.int32),
            pltpu.VMEM((S, D), jnp.float32),
            pltpu.VMEM((S, D), jnp.float32),
            pltpu.SemaphoreType.DMA,
        ],
    )
    def k(x_hbm, tok_hbm, pos_hbm, out_hbm, idx_v, rows_v, pos_v, sem):
        wid = lax.axis_index("s") * NC + lax.axis_index("c")

        pltpu.sync_copy(pos_hbm, pos_v)

        @pl.loop(0, seqs_per_tile)
        def _seq_loop(i):
            base = (wid * seqs_per_tile + i) * S
            pltpu.sync_copy(x_hbm.at[pl.ds(base, S)], idx_v)
            pltpu.async_copy(tok_hbm.at[idx_v], rows_v, sem).wait()

            @pl.loop(0, S)
            def _row_loop(r):
                for c in range(D // LANES):
                    sl = pl.ds(c * LANES, LANES)
                    rows_v[r, sl] = rows_v[r, sl] + pos_v[r, sl]

            pltpu.sync_copy(rows_v, out_hbm.at[pl.ds(base, S)])

    return k


def kernel(x, token_table, pos_table):
    B, S = x.shape
    V, D = token_table.shape
    k = _make_sc_kernel(B, S, V, D)
    out = k(x.reshape(B * S), token_table, pos_table)
    return out.reshape(B, S, D)


# speedup vs baseline: 7.4341x; 1.8781x over previous
import functools

import jax
import jax.numpy as jnp
from jax import lax
from jax.experimental import pallas as pl
from jax.experimental.pallas import tpu as pltpu
from jax.experimental.pallas import tpu_sc as plsc

NC = 2
NS = 16
LANES = 16
NBUF = 3


def _make_sc_kernel(B, S, V, D):
    n_tiles = NC * NS
    spt = B // n_tiles
    mesh = plsc.VectorSubcoreMesh(core_axis_name="c", subcore_axis_name="s")

    @functools.partial(
        pl.kernel,
        out_type=jax.ShapeDtypeStruct((B * S, D), jnp.float32),
        mesh=mesh,
        scratch_types=[
            pltpu.VMEM((spt * S,), jnp.int32),
            pltpu.VMEM((S, D), jnp.float32),
        ]
        + [pltpu.VMEM((S, D), jnp.float32)] * NBUF
        + [pltpu.SemaphoreType.DMA] * (2 * NBUF),
    )
    def k(x_hbm, tok_hbm, pos_hbm, out_hbm, idx_v, pos_v, *bufs_and_sems):
        rows = bufs_and_sems[:NBUF]
        g_sem = bufs_and_sems[NBUF:2 * NBUF]
        o_sem = bufs_and_sems[2 * NBUF:]

        wid = lax.axis_index("s") * NC + lax.axis_index("c")
        seq0 = wid * spt

        pltpu.sync_copy(pos_hbm, pos_v)
        pltpu.sync_copy(x_hbm.at[pl.ds(seq0 * S, spt * S)], idx_v)

        def g_start(t, b):
            pltpu.make_async_copy(
                tok_hbm.at[idx_v.at[pl.ds(t * S, S)]], rows[b],
                g_sem[b]).start()

        def g_wait(b):
            pltpu.make_async_copy(
                tok_hbm.at[idx_v.at[pl.ds(0, S)]], rows[b],
                g_sem[b]).wait()

        def o_start(t, b):
            pltpu.make_async_copy(
                rows[b], out_hbm.at[pl.ds((seq0 + t) * S, S)], o_sem[b]).start()

        def o_wait(b):
            pltpu.make_async_copy(
                rows[b], out_hbm.at[pl.ds(seq0 * S, S)], o_sem[b]).wait()

        def consume(t, b):
            g_wait(b)

            @pl.loop(0, S)
            def _row_loop(r):
                for c in range(D // LANES):
                    sl = pl.ds(c * LANES, LANES)
                    plsc.addupdate(rows[b].at[r, sl], pos_v[r, sl])

            o_start(t, b)

        for b in range(2):
            g_start(b, b)

        n_main = spt // NBUF * NBUF

        @pl.loop(0, n_main // NBUF)
        def _main(p):
            t0 = p * NBUF
            for b in range(NBUF):
                t = t0 + b
                consume(t, b)
                nxt = t + 2
                bn = (b + 2) % NBUF

                @pl.when(nxt < spt)
                def _(bn=bn, nxt=nxt, t=t):
                    @pl.when(t >= 1)
                    def _(bn=bn):
                        o_wait(bn)

                    g_start(nxt, bn)

        for t in range(n_main, spt):
            consume(t, t % NBUF)

        for b in range(NBUF):
            o_wait(b)

    return k


def kernel(x, token_table, pos_table):
    B, S = x.shape
    V, D = token_table.shape
    k = _make_sc_kernel(B, S, V, D)
    out = k(x.reshape(B * S), token_table, pos_table)
    return out.reshape(B, S, D)
